# Initial kernel scaffold; baseline (speedup 1.0000x reference)
import jax
import jax.numpy as jnp
from jax.experimental import pallas as pl


def kernel(x, edge_index, motif, neg_motif, rm_feat_0, rm_feat_1, k0, k1, e1W1, e1b1, e1W2, e1b2, e2W1, e2b1, e2W2, e2b2, Ws_0, Ws_1, bias_0, bias_1, mW1, mb1, mW2, mb2, gamma, beta):
    raise NotImplementedError("write your pallas kernel here")


# speedup vs baseline: 1.0296x; 1.0296x over previous
import functools

import jax
import jax.numpy as jnp
from jax.experimental import pallas as pl

_EPS = 1e-5
_TEMP = 0.5


def _mm_body(x_ref, w_ref, b_ref, o_ref, *, relu_in):
    x = x_ref[...]
    if relu_in:
        x = jnp.maximum(x, 0.0)
    o_ref[...] = (
        jnp.dot(x, w_ref[...], preferred_element_type=jnp.float32) + b_ref[...]
    )


def _mm(x, w, b, relu_in=False, blk=1000):
    n, k = x.shape
    m = w.shape[1]
    g = n // blk
    return pl.pallas_call(
        functools.partial(_mm_body, relu_in=relu_in),
        grid=(g,),
        in_specs=[
            pl.BlockSpec((blk, k), lambda i: (i, 0)),
            pl.BlockSpec((k, m), lambda i: (0, 0)),
            pl.BlockSpec((1, m), lambda i: (0, 0)),
        ],
        out_specs=pl.BlockSpec((blk, m), lambda i: (i, 0)),
        out_shape=jax.ShapeDtypeStruct((n, m), jnp.float32),
    )(x, w, b.reshape(1, -1))


def _cl_body(x1_ref, x2t_ref, colsum_ref, rowsum_ref, pos_ref, *, blk):
    i = pl.program_id(0)
    x1 = x1_ref[...]
    x2t = x2t_ref[...]
    n1 = jnp.sqrt(jnp.sum(x1 * x1, -1, keepdims=True))
    n2 = jnp.sqrt(jnp.sum(x2t * x2t, 0, keepdims=True))
    sim = jnp.exp(
        jnp.dot(x1, x2t, preferred_element_type=jnp.float32)
        / (n1 * n2 + _EPS)
        / _TEMP
    )

    @pl.when(i == 0)
    def _():
        colsum_ref[...] = jnp.zeros_like(colsum_ref)

    colsum_ref[...] += jnp.sum(sim, 0, keepdims=True)
    rowsum_ref[0, 0, :] = jnp.sum(sim, 1)
    rows = jax.lax.broadcasted_iota(jnp.int32, sim.shape, 0)
    cols = jax.lax.broadcasted_iota(jnp.int32, sim.shape, 1)
    pos_ref[0, 0, :] = jnp.sum(
        jnp.where(cols == rows + i * blk, sim, 0.0), 1
    )


def _cl_loss(x1, x2, blk=400):
    n, d = x1.shape
    g = n // blk
    colsum, rowsum, pos = pl.pallas_call(
        functools.partial(_cl_body, blk=blk),
        grid=(g,),
        in_specs=[
            pl.BlockSpec((blk, d), lambda i: (i, 0)),
            pl.BlockSpec((d, n), lambda i: (0, 0)),
        ],
        out_specs=[
            pl.BlockSpec((1, n), lambda i: (0, 0)),
            pl.BlockSpec((1, 1, blk), lambda i: (i, 0, 0)),
            pl.BlockSpec((1, 1, blk), lambda i: (i, 0, 0)),
        ],
        out_shape=[
            jax.ShapeDtypeStruct((1, n), jnp.float32),
            jax.ShapeDtypeStruct((g, 1, blk), jnp.float32),
            jax.ShapeDtypeStruct((g, 1, blk), jnp.float32),
        ],
    )(x1, x2.T)
    s0 = colsum.reshape(-1)
    s1 = rowsum.reshape(-1)
    p = pos.reshape(-1)
    l1 = -jnp.mean(jnp.log(p / (s0 - p + _EPS)))
    l2 = -jnp.mean(jnp.log(p / (s1 - p + _EPS)))
    return (l1 + l2) / 2.0


def _rm_body(f_ref, wt_ref, b_ref, k_ref, o_ref):
    k = k_ref[0, 0]
    f = f_ref[...]
    d = f.shape[-1]
    fsq = jnp.sum(f * f, -1, keepdims=True)
    xn = jnp.sqrt(fsq) + _EPS
    radius = 1.0 / jnp.sqrt(jnp.abs(k) + _EPS)
    xp = f * (0.9 * 0.5 * radius / xn)
    xpsq = jnp.sum(xp * xp, -1, keepdims=True)
    wt = wt_ref[...]
    wsq = jnp.sum(wt * wt, 0, keepdims=True)
    div = xpsq + wsq - 2.0 * jnp.dot(xp, wt, preferred_element_type=jnp.float32)
    dist = jnp.log((1.0 + k * xpsq) / (div + _EPS) + _EPS)
    o_ref[...] = jnp.exp((d - 1) * dist / 2.0) * jnp.cos(dist + b_ref[...])


def _rand_map(feat, W, bias, k, blk=1000):
    n, d = feat.shape
    m = W.shape[0]
    g = n // blk
    return pl.pallas_call(
        _rm_body,
        grid=(g,),
        in_specs=[
            pl.BlockSpec((blk, d), lambda i: (i, 0)),
            pl.BlockSpec((d, m), lambda i: (0, 0)),
            pl.BlockSpec((1, m), lambda i: (0, 0)),
            pl.BlockSpec((1, 1), lambda i: (0, 0)),
        ],
        out_specs=pl.BlockSpec((blk, m), lambda i: (i, 0)),
        out_shape=jax.ShapeDtypeStruct((n, m), jnp.float32),
    )(feat, W.T, bias.reshape(1, -1), k.reshape(1, 1))


def _bce_body(z_ref, w1_ref, b1_ref, w2_ref, b2_ref, o_ref, *, neg):
    i = pl.program_id(0)
    h = jnp.maximum(
        jnp.dot(z_ref[...], w1_ref[...], preferred_element_type=jnp.float32)
        + b1_ref[...],
        0.0,
    )
    logit = (
        jnp.dot(h, w2_ref[...], preferred_element_type=jnp.float32)
        + b2_ref[0, 0]
    )
    x = logit if neg else -logit
    sp = jnp.maximum(x, 0.0) + jnp.log1p(jnp.exp(-jnp.abs(x)))

    @pl.when(i == 0)
    def _():
        o_ref[...] = jnp.zeros_like(o_ref)

    o_ref[...] += jnp.sum(sp).reshape(1, 1)


def _bce(zf, w1, b1, w2, b2, neg, blk=1000):
    n, k = zf.shape
    m = w1.shape[1]
    g = n // blk
    tot = pl.pallas_call(
        functools.partial(_bce_body, neg=neg),
        grid=(g,),
        in_specs=[
            pl.BlockSpec((blk, k), lambda i: (i, 0)),
            pl.BlockSpec((k, m), lambda i: (0, 0)),
            pl.BlockSpec((1, m), lambda i: (0, 0)),
            pl.BlockSpec((m, 1), lambda i: (0, 0)),
            pl.BlockSpec((1, 1), lambda i: (0, 0)),
        ],
        out_specs=pl.BlockSpec((1, 1), lambda i: (0, 0)),
        out_shape=jax.ShapeDtypeStruct((1, 1), jnp.float32),
    )(zf, w1, b1.reshape(1, -1), w2, b2.reshape(1, 1))
    return tot[0, 0] / n


def _ln_body(a_ref, b_ref, c_ref, g_ref, be_ref, o_ref):
    x = jnp.concatenate([a_ref[...], b_ref[...], c_ref[...]], -1)
    mu = jnp.mean(x, -1, keepdims=True)
    var = jnp.mean((x - mu) * (x - mu), -1, keepdims=True)
    o_ref[...] = (x - mu) / jnp.sqrt(var + 1e-5) * g_ref[...] + be_ref[...]


def _layer_norm3(a, b, c, gamma, beta, blk=1000):
    n = a.shape[0]
    da, db, dc = a.shape[1], b.shape[1], c.shape[1]
    dt = da + db + dc
    g = n // blk
    return pl.pallas_call(
        _ln_body,
        grid=(g,),
        in_specs=[
            pl.BlockSpec((blk, da), lambda i: (i, 0)),
            pl.BlockSpec((blk, db), lambda i: (i, 0)),
            pl.BlockSpec((blk, dc), lambda i: (i, 0)),
            pl.BlockSpec((1, dt), lambda i: (0, 0)),
            pl.BlockSpec((1, dt), lambda i: (0, 0)),
        ],
        out_specs=pl.BlockSpec((blk, dt), lambda i: (i, 0)),
        out_shape=jax.ShapeDtypeStruct((n, dt), jnp.float32),
    )(a, b, c, gamma.reshape(1, -1), beta.reshape(1, -1))


def _gcn(x, src, dst, coef, W1, b1, W2, b2):
    n = x.shape[0]
    h = _mm(x, W1, b1)
    h = jnp.zeros_like(h).at[dst].add(h[src] * coef[:, None])
    h = _mm(h, W2, b2, relu_in=True)
    h = jnp.zeros_like(h).at[dst].add(h[src] * coef[:, None])
    return h


def kernel(x, edge_index, motif, neg_motif, rm_feat_0, rm_feat_1, k0, k1,
           e1W1, e1b1, e1W2, e1b2, e2W1, e2b1, e2W2, e2b2, Ws_0, Ws_1,
           bias_0, bias_1, mW1, mb1, mW2, mb2, gamma, beta):
    n = x.shape[0]
    ar = jnp.arange(n, dtype=edge_index.dtype)
    src = jnp.concatenate([edge_index[0], ar])
    dst = jnp.concatenate([edge_index[1], ar])
    deg = jnp.zeros((n,), jnp.float32).at[dst].add(1.0)
    dinv = 1.0 / jnp.sqrt(jnp.maximum(deg, 1.0))
    coef = dinv[src] * dinv[dst]

    xe = _gcn(x, src, dst, coef, e1W1, e1b1, e1W2, e1b2)
    z0 = _rand_map(rm_feat_0, Ws_0, bias_0, k0)
    z1 = _rand_map(rm_feat_1, Ws_1, bias_1, k1)
    emb0 = _gcn(z0, src, dst, coef, e2W1, e2b1, e2W2, e2b2)
    emb1 = _gcn(z1, src, dst, coef, e2W1, e2b1, e2W2, e2b2)

    closs = _cl_loss(xe, emb0) + _cl_loss(xe, emb1)

    mloss = 0.0
    for z in (z0, z1):
        pf = jnp.concatenate([z[motif[0]], z[motif[1]], z[motif[2]]], -1)
        nf = jnp.concatenate(
            [z[neg_motif[0]], z[neg_motif[1]], z[neg_motif[2]]], -1)
        mloss = mloss + _bce(pf, mW1, mb1, mW2, mb2, neg=False)
        mloss = mloss + _bce(nf, mW1, mb1, mW2, mb2, neg=True)

    loss = closs / 2.0 + mloss / 2.0
    out = _layer_norm3(xe, emb0, emb1, gamma, beta)
    return (out, loss)


# speedup vs baseline: 2.2132x; 2.1495x over previous
import functools

import jax
import jax.numpy as jnp
from jax.experimental import pallas as pl

_EPS = 1e-5
_TEMP = 0.5


def _mm_body(x_ref, w_ref, b_ref, o_ref, *, relu_in):
    x = x_ref[...]
    if relu_in:
        x = jnp.maximum(x, 0.0)
    o_ref[...] = (
        jnp.dot(x, w_ref[...], preferred_element_type=jnp.float32) + b_ref[...]
    )


def _mm(x, w, b, relu_in=False, blk=1000):
    n, k = x.shape
    m = w.shape[1]
    g = n // blk
    return pl.pallas_call(
        functools.partial(_mm_body, relu_in=relu_in),
        grid=(g,),
        in_specs=[
            pl.BlockSpec((blk, k), lambda i: (i, 0)),
            pl.BlockSpec((k, m), lambda i: (0, 0)),
            pl.BlockSpec((1, m), lambda i: (0, 0)),
        ],
        out_specs=pl.BlockSpec((blk, m), lambda i: (i, 0)),
        out_shape=jax.ShapeDtypeStruct((n, m), jnp.float32),
    )(x, w, b.reshape(1, -1))


def _cl_body(x1_ref, x2t_ref, colsum_ref, rowsum_ref, pos_ref, *, blk):
    i = pl.program_id(0)
    x1 = x1_ref[...]
    x2t = x2t_ref[...]
    n1 = jnp.sqrt(jnp.sum(x1 * x1, -1, keepdims=True))
    n2 = jnp.sqrt(jnp.sum(x2t * x2t, 0, keepdims=True))
    sim = jnp.exp(
        jnp.dot(x1, x2t, preferred_element_type=jnp.float32)
        / (n1 * n2 + _EPS)
        / _TEMP
    )

    @pl.when(i == 0)
    def _():
        colsum_ref[...] = jnp.zeros_like(colsum_ref)

    colsum_ref[...] += jnp.sum(sim, 0, keepdims=True)
    rowsum_ref[0, 0, :] = jnp.sum(sim, 1)
    rows = jax.lax.broadcasted_iota(jnp.int32, sim.shape, 0)
    cols = jax.lax.broadcasted_iota(jnp.int32, sim.shape, 1)
    pos_ref[0, 0, :] = jnp.sum(
        jnp.where(cols == rows + i * blk, sim, 0.0), 1
    )


def _cl_loss(x1, x2, blk=400):
    n, d = x1.shape
    g = n // blk
    colsum, rowsum, pos = pl.pallas_call(
        functools.partial(_cl_body, blk=blk),
        grid=(g,),
        in_specs=[
            pl.BlockSpec((blk, d), lambda i: (i, 0)),
            pl.BlockSpec((d, n), lambda i: (0, 0)),
        ],
        out_specs=[
            pl.BlockSpec((1, n), lambda i: (0, 0)),
            pl.BlockSpec((1, 1, blk), lambda i: (i, 0, 0)),
            pl.BlockSpec((1, 1, blk), lambda i: (i, 0, 0)),
        ],
        out_shape=[
            jax.ShapeDtypeStruct((1, n), jnp.float32),
            jax.ShapeDtypeStruct((g, 1, blk), jnp.float32),
            jax.ShapeDtypeStruct((g, 1, blk), jnp.float32),
        ],
    )(x1, x2.T)
    s0 = colsum.reshape(-1)
    s1 = rowsum.reshape(-1)
    p = pos.reshape(-1)
    l1 = -jnp.mean(jnp.log(p / (s0 - p + _EPS)))
    l2 = -jnp.mean(jnp.log(p / (s1 - p + _EPS)))
    return (l1 + l2) / 2.0


def _rm_body(f_ref, wt_ref, b_ref, k_ref, o_ref):
    k = k_ref[0, 0]
    f = f_ref[...]
    d = f.shape[-1]
    fsq = jnp.sum(f * f, -1, keepdims=True)
    xn = jnp.sqrt(fsq) + _EPS
    radius = 1.0 / jnp.sqrt(jnp.abs(k) + _EPS)
    xp = f * (0.9 * 0.5 * radius / xn)
    xpsq = jnp.sum(xp * xp, -1, keepdims=True)
    wt = wt_ref[...]
    wsq = jnp.sum(wt * wt, 0, keepdims=True)
    div = xpsq + wsq - 2.0 * jnp.dot(xp, wt, preferred_element_type=jnp.float32)
    dist = jnp.log((1.0 + k * xpsq) / (div + _EPS) + _EPS)
    o_ref[...] = jnp.exp((d - 1) * dist / 2.0) * jnp.cos(dist + b_ref[...])


def _rand_map(feat, W, bias, k, blk=1000):
    n, d = feat.shape
    m = W.shape[0]
    g = n // blk
    return pl.pallas_call(
        _rm_body,
        grid=(g,),
        in_specs=[
            pl.BlockSpec((blk, d), lambda i: (i, 0)),
            pl.BlockSpec((d, m), lambda i: (0, 0)),
            pl.BlockSpec((1, m), lambda i: (0, 0)),
            pl.BlockSpec((1, 1), lambda i: (0, 0)),
        ],
        out_specs=pl.BlockSpec((blk, m), lambda i: (i, 0)),
        out_shape=jax.ShapeDtypeStruct((n, m), jnp.float32),
    )(feat, W.T, bias.reshape(1, -1), k.reshape(1, 1))


def _bce_body(z_ref, w1_ref, b1_ref, w2_ref, b2_ref, o_ref, *, neg):
    i = pl.program_id(0)
    h = jnp.maximum(
        jnp.dot(z_ref[...], w1_ref[...], preferred_element_type=jnp.float32)
        + b1_ref[...],
        0.0,
    )
    logit = (
        jnp.dot(h, w2_ref[...], preferred_element_type=jnp.float32)
        + b2_ref[0, 0]
    )
    x = logit if neg else -logit
    sp = jnp.maximum(x, 0.0) + jnp.log1p(jnp.exp(-jnp.abs(x)))

    @pl.when(i == 0)
    def _():
        o_ref[...] = jnp.zeros_like(o_ref)

    o_ref[...] += jnp.sum(sp).reshape(1, 1)


def _bce(zf, w1, b1, w2, b2, neg, blk=1000):
    n, k = zf.shape
    m = w1.shape[1]
    g = n // blk
    tot = pl.pallas_call(
        functools.partial(_bce_body, neg=neg),
        grid=(g,),
        in_specs=[
            pl.BlockSpec((blk, k), lambda i: (i, 0)),
            pl.BlockSpec((k, m), lambda i: (0, 0)),
            pl.BlockSpec((1, m), lambda i: (0, 0)),
            pl.BlockSpec((m, 1), lambda i: (0, 0)),
            pl.BlockSpec((1, 1), lambda i: (0, 0)),
        ],
        out_specs=pl.BlockSpec((1, 1), lambda i: (0, 0)),
        out_shape=jax.ShapeDtypeStruct((1, 1), jnp.float32),
    )(zf, w1, b1.reshape(1, -1), w2, b2.reshape(1, 1))
    return tot[0, 0] / n


def _ln_body(a_ref, b_ref, c_ref, g_ref, be_ref, o_ref):
    x = jnp.concatenate([a_ref[...], b_ref[...], c_ref[...]], -1)
    mu = jnp.mean(x, -1, keepdims=True)
    var = jnp.mean((x - mu) * (x - mu), -1, keepdims=True)
    o_ref[...] = (x - mu) / jnp.sqrt(var + 1e-5) * g_ref[...] + be_ref[...]


def _layer_norm3(a, b, c, gamma, beta, blk=1000):
    n = a.shape[0]
    da, db, dc = a.shape[1], b.shape[1], c.shape[1]
    dt = da + db + dc
    g = n // blk
    return pl.pallas_call(
        _ln_body,
        grid=(g,),
        in_specs=[
            pl.BlockSpec((blk, da), lambda i: (i, 0)),
            pl.BlockSpec((blk, db), lambda i: (i, 0)),
            pl.BlockSpec((blk, dc), lambda i: (i, 0)),
            pl.BlockSpec((1, dt), lambda i: (0, 0)),
            pl.BlockSpec((1, dt), lambda i: (0, 0)),
        ],
        out_specs=pl.BlockSpec((blk, dt), lambda i: (i, 0)),
        out_shape=jax.ShapeDtypeStruct((n, dt), jnp.float32),
    )(a, b, c, gamma.reshape(1, -1), beta.reshape(1, -1))


def _spmm_body(a_ref, h_ref, o_ref):
    o_ref[...] = jnp.dot(
        a_ref[...], h_ref[...], preferred_element_type=jnp.float32
    )


def _spmm_fused_body(a_ref, u_ref, w2_ref, b2_ref, o_ref):
    t = jnp.dot(a_ref[...], u_ref[...], preferred_element_type=jnp.float32)
    t = jnp.maximum(t, 0.0)
    o_ref[...] = (
        jnp.dot(t, w2_ref[...], preferred_element_type=jnp.float32)
        + b2_ref[...]
    )


def _agg(A, h, blk=400):
    n = A.shape[0]
    f = h.shape[1]
    return pl.pallas_call(
        _spmm_body,
        grid=(n // blk,),
        in_specs=[
            pl.BlockSpec((blk, n), lambda i: (i, 0)),
            pl.BlockSpec((n, f), lambda i: (0, 0)),
        ],
        out_specs=pl.BlockSpec((blk, f), lambda i: (i, 0)),
        out_shape=jax.ShapeDtypeStruct((n, f), jnp.float32),
    )(A, h)


def _agg_relu_mm(A, u, w2, b2, blk=400):
    n = A.shape[0]
    k = u.shape[1]
    m = w2.shape[1]
    return pl.pallas_call(
        _spmm_fused_body,
        grid=(n // blk,),
        in_specs=[
            pl.BlockSpec((blk, n), lambda i: (i, 0)),
            pl.BlockSpec((n, k), lambda i: (0, 0)),
            pl.BlockSpec((k, m), lambda i: (0, 0)),
            pl.BlockSpec((1, m), lambda i: (0, 0)),
        ],
        out_specs=pl.BlockSpec((blk, m), lambda i: (i, 0)),
        out_shape=jax.ShapeDtypeStruct((n, m), jnp.float32),
    )(A, u, w2, b2.reshape(1, -1))


def _gcn(x, A, W1, b1, W2, b2):
    u = _mm(x, W1, b1)
    h2 = _agg_relu_mm(A, u, W2, b2)
    return _agg(A, h2)


def kernel(x, edge_index, motif, neg_motif, rm_feat_0, rm_feat_1, k0, k1,
           e1W1, e1b1, e1W2, e1b2, e2W1, e2b1, e2W2, e2b2, Ws_0, Ws_1,
           bias_0, bias_1, mW1, mb1, mW2, mb2, gamma, beta):
    n = x.shape[0]
    ar = jnp.arange(n, dtype=edge_index.dtype)
    src = jnp.concatenate([edge_index[0], ar])
    dst = jnp.concatenate([edge_index[1], ar])
    deg = jnp.zeros((n,), jnp.float32).at[dst].add(1.0)
    dinv = 1.0 / jnp.sqrt(jnp.maximum(deg, 1.0))
    coef = dinv[src] * dinv[dst]
    A = (
        jnp.zeros((n * n,), jnp.float32)
        .at[dst * n + src]
        .add(coef)
        .reshape(n, n)
    )

    xe = _gcn(x, A, e1W1, e1b1, e1W2, e1b2)
    z0 = _rand_map(rm_feat_0, Ws_0, bias_0, k0)
    z1 = _rand_map(rm_feat_1, Ws_1, bias_1, k1)
    emb0 = _gcn(z0, A, e2W1, e2b1, e2W2, e2b2)
    emb1 = _gcn(z1, A, e2W1, e2b1, e2W2, e2b2)

    closs = _cl_loss(xe, emb0) + _cl_loss(xe, emb1)

    mloss = 0.0
    for z in (z0, z1):
        pf = jnp.concatenate([z[motif[0]], z[motif[1]], z[motif[2]]], -1)
        nf = jnp.concatenate(
            [z[neg_motif[0]], z[neg_motif[1]], z[neg_motif[2]]], -1)
        mloss = mloss + _bce(pf, mW1, mb1, mW2, mb2, neg=False)
        mloss = mloss + _bce(nf, mW1, mb1, mW2, mb2, neg=True)

    loss = closs / 2.0 + mloss / 2.0
    out = _layer_norm3(xe, emb0, emb1, gamma, beta)
    return (out, loss)
